# baseline (speedup 1.0000x reference)
import functools

import jax
import jax.numpy as jnp
from jax import lax
from jax.experimental import pallas as pl
from jax.experimental.pallas import tpu as pltpu
from jax.experimental.pallas import tpu_sc as plsc

_DIM = 128
_VOCAB = 300
_BATCH = 16384

_info = plsc.get_sparse_core_info()
_NC, _NS = _info.num_cores, _info.num_subcores
_NW = _NC * _NS
_B_PER_W = _BATCH // _NW
_CHUNK = 128
_NCHUNK = _B_PER_W // _CHUNK


@functools.partial(
    pl.kernel,
    mesh=plsc.VectorSubcoreMesh(core_axis_name="c", subcore_axis_name="s"),
    out_type=jax.ShapeDtypeStruct((_BATCH, _DIM), jnp.float32),
    scratch_types=[
        pltpu.VMEM((_B_PER_W,), jnp.int32),
        pltpu.VMEM((_CHUNK, _DIM), jnp.float32),
        pltpu.VMEM((_CHUNK, _DIM), jnp.float32),
        pltpu.SemaphoreType.DMA,
        pltpu.SemaphoreType.DMA,
    ],
)
def _embed_kernel(idx_hbm, table_hbm, out_hbm, idx_v, buf0, buf1, sem0, sem1):
    wid = lax.axis_index("s") * _NC + lax.axis_index("c")
    base = wid * _B_PER_W
    pltpu.sync_copy(idx_hbm.at[pl.ds(base, _B_PER_W)], idx_v)

    bufs = (buf0, buf1)
    sems = (sem0, sem1)

    def start_gather(k):
        return pltpu.async_copy(
            table_hbm.at[idx_v.at[pl.ds(k * _CHUNK, _CHUNK)]],
            bufs[k % 2],
            sems[k % 2],
        )

    handles = [None] * _NCHUNK
    handles[0] = start_gather(0)
    for k in range(_NCHUNK):
        if k + 1 < _NCHUNK:
            handles[k + 1] = start_gather(k + 1)
        handles[k].wait()
        pltpu.sync_copy(bufs[k % 2], out_hbm.at[pl.ds(base + k * _CHUNK, _CHUNK)])


def kernel(t, table):
    return _embed_kernel(t.astype(jnp.int32), table)


# speedup vs baseline: 1.3549x; 1.3549x over previous
import functools

import jax
import jax.numpy as jnp
from jax import lax
from jax.experimental import pallas as pl
from jax.experimental.pallas import tpu as pltpu
from jax.experimental.pallas import tpu_sc as plsc

_DIM = 128
_VOCAB = 300
_BATCH = 16384

_info = plsc.get_sparse_core_info()
_NC, _NS = _info.num_cores, _info.num_subcores
_NW = _NC * _NS
_B_PER_W = _BATCH // _NW
_CHUNK = 128
_NCHUNK = _B_PER_W // _CHUNK


@functools.partial(
    pl.kernel,
    mesh=plsc.VectorSubcoreMesh(core_axis_name="c", subcore_axis_name="s"),
    out_type=jax.ShapeDtypeStruct((_BATCH, _DIM), jnp.float32),
    scratch_types=[
        pltpu.VMEM_SHARED((_VOCAB, _DIM), jnp.float32),
        pltpu.VMEM((_B_PER_W,), jnp.int32),
        pltpu.VMEM((_CHUNK, _DIM), jnp.float32),
        pltpu.VMEM((_CHUNK, _DIM), jnp.float32),
        pltpu.SemaphoreType.DMA,
        pltpu.SemaphoreType.DMA,
        pltpu.SemaphoreType.DMA,
        pltpu.SemaphoreType.DMA,
    ],
)
def _embed_kernel(idx_hbm, table_hbm, out_hbm, table_sh, idx_v, buf0, buf1,
                  gsem0, gsem1, ssem0, ssem1):
    sid = lax.axis_index("s")
    wid = sid * _NC + lax.axis_index("c")
    base = wid * _B_PER_W

    @pl.when(sid == 0)
    def _stage_table():
        pltpu.sync_copy(table_hbm, table_sh)

    pltpu.sync_copy(idx_hbm.at[pl.ds(base, _B_PER_W)], idx_v)
    plsc.subcore_barrier()

    bufs = (buf0, buf1)
    gsems = (gsem0, gsem1)
    ssems = (ssem0, ssem1)

    def start_gather(k):
        return pltpu.async_copy(
            table_sh.at[idx_v.at[pl.ds(k * _CHUNK, _CHUNK)]],
            bufs[k % 2],
            gsems[k % 2],
        )

    def start_store(k):
        return pltpu.async_copy(
            bufs[k % 2],
            out_hbm.at[pl.ds(base + k * _CHUNK, _CHUNK)],
            ssems[k % 2],
        )

    gathers = [None] * _NCHUNK
    stores = [None] * _NCHUNK
    gathers[0] = start_gather(0)
    for k in range(_NCHUNK):
        if k + 1 < _NCHUNK:
            if k - 1 >= 0:
                stores[k - 1].wait()
            gathers[k + 1] = start_gather(k + 1)
        gathers[k].wait()
        stores[k] = start_store(k)
    stores[_NCHUNK - 2].wait()
    stores[_NCHUNK - 1].wait()


def kernel(t, table):
    return _embed_kernel(t.astype(jnp.int32), table)


# speedup vs baseline: 1.3722x; 1.0127x over previous
import functools

import jax
import jax.numpy as jnp
from jax import lax
from jax.experimental import pallas as pl
from jax.experimental.pallas import tpu as pltpu
from jax.experimental.pallas import tpu_sc as plsc

_DIM = 128
_VOCAB = 300
_BATCH = 16384

_info = plsc.get_sparse_core_info()
_NC, _NS = _info.num_cores, _info.num_subcores
_NW = _NC * _NS
_B_PER_W = _BATCH // _NW
_CHUNK = 128
_NCHUNK = _B_PER_W // _CHUNK


@functools.partial(
    pl.kernel,
    mesh=plsc.VectorSubcoreMesh(core_axis_name="c", subcore_axis_name="s"),
    out_type=jax.ShapeDtypeStruct((_BATCH, _DIM), jnp.float32),
    scratch_types=[
        pltpu.VMEM_SHARED((_VOCAB, _DIM), jnp.float32),
        pltpu.VMEM((_B_PER_W,), jnp.int32),
    ] + [pltpu.VMEM((_CHUNK, _DIM), jnp.float32) for _ in range(_NCHUNK)]
      + [pltpu.SemaphoreType.DMA for _ in range(2 * _NCHUNK)],
)
def _embed_kernel(idx_hbm, table_hbm, out_hbm, table_sh, idx_v, *bufs_sems):
    bufs = bufs_sems[:_NCHUNK]
    gsems = bufs_sems[_NCHUNK:2 * _NCHUNK]
    ssems = bufs_sems[2 * _NCHUNK:]
    sid = lax.axis_index("s")
    wid = sid * _NC + lax.axis_index("c")
    base = wid * _B_PER_W

    @pl.when(sid == 0)
    def _stage_table():
        pltpu.sync_copy(table_hbm, table_sh)

    pltpu.sync_copy(idx_hbm.at[pl.ds(base, _B_PER_W)], idx_v)
    plsc.subcore_barrier()

    gathers = [
        pltpu.async_copy(
            table_sh.at[idx_v.at[pl.ds(k * _CHUNK, _CHUNK)]],
            bufs[k],
            gsems[k],
        )
        for k in range(_NCHUNK)
    ]
    stores = []
    for k in range(_NCHUNK):
        gathers[k].wait()
        stores.append(pltpu.async_copy(
            bufs[k],
            out_hbm.at[pl.ds(base + k * _CHUNK, _CHUNK)],
            ssems[k],
        ))
    for st in stores:
        st.wait()


def kernel(t, table):
    return _embed_kernel(t.astype(jnp.int32), table)


# speedup vs baseline: 1.3769x; 1.0035x over previous
import functools

import jax
import jax.numpy as jnp
from jax import lax
from jax.experimental import pallas as pl
from jax.experimental.pallas import tpu as pltpu
from jax.experimental.pallas import tpu_sc as plsc

_DIM = 128
_VOCAB = 300
_BATCH = 16384

_info = plsc.get_sparse_core_info()
_NC, _NS = _info.num_cores, _info.num_subcores
_NW = _NC * _NS
_B_PER_W = _BATCH // _NW
_CHUNK = 64
_NCHUNK = _B_PER_W // _CHUNK


@functools.partial(
    pl.kernel,
    mesh=plsc.VectorSubcoreMesh(core_axis_name="c", subcore_axis_name="s"),
    out_type=jax.ShapeDtypeStruct((_BATCH, _DIM), jnp.float32),
    scratch_types=[
        pltpu.VMEM_SHARED((_VOCAB, _DIM), jnp.float32),
        pltpu.VMEM((_B_PER_W,), jnp.int32),
    ] + [pltpu.VMEM((_CHUNK, _DIM), jnp.float32) for _ in range(_NCHUNK)]
      + [pltpu.SemaphoreType.DMA for _ in range(2 * _NCHUNK)],
)
def _embed_kernel(idx_hbm, table_hbm, out_hbm, table_sh, idx_v, *bufs_sems):
    bufs = bufs_sems[:_NCHUNK]
    gsems = bufs_sems[_NCHUNK:2 * _NCHUNK]
    ssems = bufs_sems[2 * _NCHUNK:]
    sid = lax.axis_index("s")
    wid = sid * _NC + lax.axis_index("c")
    base = wid * _B_PER_W

    @pl.when(sid == 0)
    def _stage_table():
        pltpu.sync_copy(table_hbm, table_sh)

    pltpu.sync_copy(idx_hbm.at[pl.ds(base, _B_PER_W)], idx_v)
    plsc.subcore_barrier()

    gathers = [
        pltpu.async_copy(
            table_sh.at[idx_v.at[pl.ds(k * _CHUNK, _CHUNK)]],
            bufs[k],
            gsems[k],
        )
        for k in range(_NCHUNK)
    ]
    stores = []
    for k in range(_NCHUNK):
        gathers[k].wait()
        stores.append(pltpu.async_copy(
            bufs[k],
            out_hbm.at[pl.ds(base + k * _CHUNK, _CHUNK)],
            ssems[k],
        ))
    for st in stores:
        st.wait()


def kernel(t, table):
    return _embed_kernel(t.astype(jnp.int32), table)


# speedup vs baseline: 1.4243x; 1.0344x over previous
import functools

import jax
import jax.numpy as jnp
from jax import lax
from jax.experimental import pallas as pl
from jax.experimental.pallas import tpu as pltpu
from jax.experimental.pallas import tpu_sc as plsc

_DIM = 128
_VOCAB = 300
_BATCH = 16384

_info = plsc.get_sparse_core_info()
_NC, _NS = _info.num_cores, _info.num_subcores
_NW = _NC * _NS
_B_PER_W = _BATCH // _NW
_CHUNK = 64
_NCHUNK = _B_PER_W // _CHUNK


@functools.partial(
    pl.kernel,
    mesh=plsc.VectorSubcoreMesh(core_axis_name="c", subcore_axis_name="s"),
    out_type=jax.ShapeDtypeStruct((_BATCH, _DIM), jnp.float32),
    scratch_types=[
        pltpu.VMEM_SHARED((_VOCAB, _DIM), jnp.float32),
        pltpu.VMEM((_B_PER_W,), jnp.int32),
        pltpu.SemaphoreType.DMA,
    ] + [pltpu.VMEM((_CHUNK, _DIM), jnp.float32) for _ in range(_NCHUNK)]
      + [pltpu.SemaphoreType.DMA for _ in range(2 * _NCHUNK)],
)
def _embed_kernel(idx_hbm, table_hbm, out_hbm, table_sh, idx_v, isem, *bufs_sems):
    bufs = bufs_sems[:_NCHUNK]
    gsems = bufs_sems[_NCHUNK:2 * _NCHUNK]
    ssems = bufs_sems[2 * _NCHUNK:]
    sid = lax.axis_index("s")
    wid = sid * _NC + lax.axis_index("c")
    base = wid * _B_PER_W

    idx_cp = pltpu.async_copy(idx_hbm.at[pl.ds(base, _B_PER_W)], idx_v, isem)
    rows_per_tile = 24
    full_tiles = _VOCAB // rows_per_tile
    last_rows = _VOCAB - rows_per_tile * full_tiles

    @pl.when(sid < full_tiles)
    def _stage_table():
        pltpu.sync_copy(
            table_hbm.at[pl.ds(sid * rows_per_tile, rows_per_tile)],
            table_sh.at[pl.ds(sid * rows_per_tile, rows_per_tile)],
        )

    @pl.when(sid == full_tiles)
    def _stage_table_last():
        pltpu.sync_copy(
            table_hbm.at[pl.ds(full_tiles * rows_per_tile, last_rows)],
            table_sh.at[pl.ds(full_tiles * rows_per_tile, last_rows)],
        )

    plsc.subcore_barrier()
    idx_cp.wait()

    gathers = [
        pltpu.async_copy(
            table_sh.at[idx_v.at[pl.ds(k * _CHUNK, _CHUNK)]],
            bufs[k],
            gsems[k],
        )
        for k in range(_NCHUNK)
    ]
    stores = []
    for k in range(_NCHUNK):
        gathers[k].wait()
        stores.append(pltpu.async_copy(
            bufs[k],
            out_hbm.at[pl.ds(base + k * _CHUNK, _CHUNK)],
            ssems[k],
        ))
    for st in stores:
        st.wait()


def kernel(t, table):
    return _embed_kernel(t.astype(jnp.int32), table)
